# baseline (speedup 1.0000x reference)
import functools

import jax
import jax.numpy as jnp
from jax.experimental import pallas as pl
from jax.experimental.pallas import tpu as pltpu

_NEG_INF = -3.0e38


def _prologue_body(x_ref, w_ref, b_ref, s2w_ref, xp_ref, s2_ref, cmax_ref):
    t = pl.program_id(0)
    xp = jax.lax.dot_general(
        x_ref[...], w_ref[...],
        dimension_numbers=(((1,), (1,)), ((), ())),
        preferred_element_type=jnp.float32,
    ) + b_ref[...]
    xp_ref[...] = xp
    s2 = jnp.sum(xp * s2w_ref[...], axis=1, keepdims=True)
    s2_ref[...] = s2

    @pl.when(t == 0)
    def _():
        cmax_ref[...] = jnp.full((1, 1), _NEG_INF, jnp.float32)

    cmax_ref[...] = jnp.maximum(cmax_ref[...],
                                jnp.max(s2, axis=(0, 1), keepdims=True))


def _vbuild_body(xp_ref, s2_ref, cmax_ref, vc_ref):
    e = jnp.exp(s2_ref[...] - cmax_ref[...])
    v = xp_ref[...] * e
    t, f = v.shape
    vc_ref[...] = jnp.concatenate(
        [v, e, jnp.zeros((t, f - 1), jnp.float32)], axis=1)


def _main_body(adj_hbm, vc_ref, vself_ref, out_ref, abuf, sems, *,
               out_f, tsub, nbuf, nchunks):
    i = pl.program_id(0)

    def _start(chunk, slot):
        pltpu.make_async_copy(
            adj_hbm.at[pl.ds(chunk * tsub, tsub), :],
            abuf.at[slot],
            sems.at[slot],
        ).start()

    @pl.when(i == 0)
    def _():
        for b in range(nbuf):
            _start(b, b)

    slot = jax.lax.rem(i, nbuf)
    pltpu.make_async_copy(
        adj_hbm.at[pl.ds(i * tsub, tsub), :],
        abuf.at[slot],
        sems.at[slot],
    ).wait()
    res = jnp.dot(abuf[slot], vc_ref[...],
                  preferred_element_type=jnp.float32)
    num = res[:, :out_f] + vself_ref[:, :out_f]
    den = res[:, out_f:out_f + 1] + vself_ref[:, out_f:out_f + 1]
    out_ref[...] = jax.nn.sigmoid(num / den)

    @pl.when(i + nbuf < nchunks)
    def _():
        _start(i + nbuf, slot)


def kernel(X, adj, W, b, S):
    n, in_f = X.shape
    out_f = W.shape[0]

    tp = 1000
    tsub = 80
    nbuf = 8
    nchunks = n // tsub

    s2w = S[out_f:].reshape(1, out_f)
    b2 = b.reshape(1, out_f)

    xp, s2, cmax = pl.pallas_call(
        _prologue_body,
        grid=(n // tp,),
        in_specs=[
            pl.BlockSpec((tp, in_f), lambda t: (t, 0)),
            pl.BlockSpec((out_f, in_f), lambda t: (0, 0)),
            pl.BlockSpec((1, out_f), lambda t: (0, 0)),
            pl.BlockSpec((1, out_f), lambda t: (0, 0)),
        ],
        out_specs=[
            pl.BlockSpec((tp, out_f), lambda t: (t, 0)),
            pl.BlockSpec((tp, 1), lambda t: (t, 0)),
            pl.BlockSpec((1, 1), lambda t: (0, 0)),
        ],
        out_shape=[
            jax.ShapeDtypeStruct((n, out_f), jnp.float32),
            jax.ShapeDtypeStruct((n, 1), jnp.float32),
            jax.ShapeDtypeStruct((1, 1), jnp.float32),
        ],
    )(X, W, b2, s2w)

    vc = pl.pallas_call(
        _vbuild_body,
        grid=(n // tp,),
        in_specs=[
            pl.BlockSpec((tp, out_f), lambda t: (t, 0)),
            pl.BlockSpec((tp, 1), lambda t: (t, 0)),
            pl.BlockSpec((1, 1), lambda t: (0, 0)),
        ],
        out_specs=pl.BlockSpec((tp, 2 * out_f), lambda t: (t, 0)),
        out_shape=jax.ShapeDtypeStruct((n, 2 * out_f), jnp.float32),
    )(xp, s2, cmax)

    out = pl.pallas_call(
        functools.partial(_main_body, out_f=out_f, tsub=tsub, nbuf=nbuf,
                          nchunks=nchunks),
        grid=(nchunks,),
        in_specs=[
            pl.BlockSpec(memory_space=pl.ANY),
            pl.BlockSpec((n, 2 * out_f), lambda i: (0, 0)),
            pl.BlockSpec((tsub, 2 * out_f), lambda i: (i, 0)),
        ],
        out_specs=pl.BlockSpec((tsub, out_f), lambda i: (i, 0)),
        out_shape=jax.ShapeDtypeStruct((n, out_f), jnp.float32),
        scratch_shapes=[
            pltpu.VMEM((nbuf, tsub, n), jnp.float32),
            pltpu.SemaphoreType.DMA((nbuf,)),
        ],
    )(adj, vc, vc)

    return out


# speedup vs baseline: 1.4190x; 1.4190x over previous
import functools

import jax
import jax.numpy as jnp
from jax.experimental import pallas as pl
from jax.experimental.pallas import tpu as pltpu

_NEG_INF = -3.0e38


def _prologue_body(x_ref, w_ref, b_ref, s2w_ref, xp_ref, s2_ref, cmax_ref):
    t = pl.program_id(0)
    xp = jax.lax.dot_general(
        x_ref[...], w_ref[...],
        dimension_numbers=(((1,), (1,)), ((), ())),
        preferred_element_type=jnp.float32,
    ) + b_ref[...]
    xp_ref[...] = xp
    s2 = jnp.sum(xp * s2w_ref[...], axis=1, keepdims=True)
    s2_ref[...] = s2

    @pl.when(t == 0)
    def _():
        cmax_ref[...] = jnp.full((1, 1), _NEG_INF, jnp.float32)

    cmax_ref[...] = jnp.maximum(cmax_ref[...],
                                jnp.max(s2, axis=(0, 1), keepdims=True))


def _vbuild_body(xp_ref, s2_ref, cmax_ref, vc_ref):
    e = jnp.exp(s2_ref[...] - cmax_ref[...])
    v = xp_ref[...] * e
    t, f = v.shape
    vc_ref[...] = jnp.concatenate(
        [v, e, jnp.zeros((t, f - 1), jnp.float32)], axis=1)


def _main_body(adj_hbm, vc_ref, vself_ref, out_ref, abuf, sems, *,
               out_f, ti, splits, nchunks):
    i = pl.program_id(0)
    tsub = ti // splits

    def _copies(chunk, buf):
        return [
            pltpu.make_async_copy(
                adj_hbm.at[pl.ds(chunk * ti + s * tsub, tsub), :],
                abuf.at[buf, pl.ds(s * tsub, tsub), :],
                sems.at[buf, s],
            )
            for s in range(splits)
        ]

    @pl.when(i == 0)
    def _():
        for c in _copies(0, 0):
            c.start()

    buf = jax.lax.rem(i, 2)
    nxt = jax.lax.rem(i + 1, 2)

    @pl.when(i + 1 < nchunks)
    def _():
        for c in _copies(i + 1, nxt):
            c.start()

    for c in _copies(i, buf):
        c.wait()

    res = jnp.dot(abuf[buf], vc_ref[...],
                  preferred_element_type=jnp.float32)
    num = res[:, :out_f] + vself_ref[:, :out_f]
    den = res[:, out_f:out_f + 1] + vself_ref[:, out_f:out_f + 1]
    out_ref[...] = jax.nn.sigmoid(num / den)


def kernel(X, adj, W, b, S):
    n, in_f = X.shape
    out_f = W.shape[0]

    tp = 1000
    ti = 400
    splits = 10
    nchunks = n // ti

    s2w = S[out_f:].reshape(1, out_f)
    b2 = b.reshape(1, out_f)

    xp, s2, cmax = pl.pallas_call(
        _prologue_body,
        grid=(n // tp,),
        in_specs=[
            pl.BlockSpec((tp, in_f), lambda t: (t, 0)),
            pl.BlockSpec((out_f, in_f), lambda t: (0, 0)),
            pl.BlockSpec((1, out_f), lambda t: (0, 0)),
            pl.BlockSpec((1, out_f), lambda t: (0, 0)),
        ],
        out_specs=[
            pl.BlockSpec((tp, out_f), lambda t: (t, 0)),
            pl.BlockSpec((tp, 1), lambda t: (t, 0)),
            pl.BlockSpec((1, 1), lambda t: (0, 0)),
        ],
        out_shape=[
            jax.ShapeDtypeStruct((n, out_f), jnp.float32),
            jax.ShapeDtypeStruct((n, 1), jnp.float32),
            jax.ShapeDtypeStruct((1, 1), jnp.float32),
        ],
    )(X, W, b2, s2w)

    vc = pl.pallas_call(
        _vbuild_body,
        grid=(n // tp,),
        in_specs=[
            pl.BlockSpec((tp, out_f), lambda t: (t, 0)),
            pl.BlockSpec((tp, 1), lambda t: (t, 0)),
            pl.BlockSpec((1, 1), lambda t: (0, 0)),
        ],
        out_specs=pl.BlockSpec((tp, 2 * out_f), lambda t: (t, 0)),
        out_shape=jax.ShapeDtypeStruct((n, 2 * out_f), jnp.float32),
    )(xp, s2, cmax)

    out = pl.pallas_call(
        functools.partial(_main_body, out_f=out_f, ti=ti, splits=splits,
                          nchunks=nchunks),
        grid=(nchunks,),
        in_specs=[
            pl.BlockSpec(memory_space=pl.ANY),
            pl.BlockSpec((n, 2 * out_f), lambda i: (0, 0)),
            pl.BlockSpec((ti, 2 * out_f), lambda i: (i, 0)),
        ],
        out_specs=pl.BlockSpec((ti, out_f), lambda i: (i, 0)),
        out_shape=jax.ShapeDtypeStruct((n, out_f), jnp.float32),
        scratch_shapes=[
            pltpu.VMEM((2, ti, n), jnp.float32),
            pltpu.SemaphoreType.DMA((2, splits)),
        ],
    )(adj, vc, vc)

    return out


# speedup vs baseline: 1.4197x; 1.0005x over previous
import functools

import jax
import jax.numpy as jnp
from jax.experimental import pallas as pl
from jax.experimental.pallas import tpu as pltpu

_NEG_INF = -3.0e38


def _prologue_body(x_ref, w_ref, b_ref, s2w_ref, xp_ref, s2_ref, cmax_ref):
    t = pl.program_id(0)
    xp = jax.lax.dot_general(
        x_ref[...], w_ref[...],
        dimension_numbers=(((1,), (1,)), ((), ())),
        preferred_element_type=jnp.float32,
    ) + b_ref[...]
    xp_ref[...] = xp
    s2 = jnp.sum(xp * s2w_ref[...], axis=1, keepdims=True)
    s2_ref[...] = s2

    @pl.when(t == 0)
    def _():
        cmax_ref[...] = jnp.full((1, 1), _NEG_INF, jnp.float32)

    cmax_ref[...] = jnp.maximum(cmax_ref[...],
                                jnp.max(s2, axis=(0, 1), keepdims=True))


def _vbuild_body(xp_ref, s2_ref, cmax_ref, vc_ref):
    e = jnp.exp(s2_ref[...] - cmax_ref[...])
    v = xp_ref[...] * e
    t, f = v.shape
    vc_ref[...] = jnp.concatenate(
        [v, e, jnp.zeros((t, f - 1), jnp.float32)], axis=1)


def _main_body(adj_hbm, vc_ref, vself_ref, out_ref, abuf, sems, *,
               out_f, ti, splits, nchunks):
    i = pl.program_id(0)
    tsub = ti // splits

    def _copies(chunk, buf):
        return [
            pltpu.make_async_copy(
                adj_hbm.at[pl.ds(chunk * ti + s * tsub, tsub), :],
                abuf.at[buf, pl.ds(s * tsub, tsub), :],
                sems.at[buf, s],
            )
            for s in range(splits)
        ]

    @pl.when(i == 0)
    def _():
        for c in _copies(0, 0):
            c.start()

    buf = jax.lax.rem(i, 2)
    nxt = jax.lax.rem(i + 1, 2)

    @pl.when(i + 1 < nchunks)
    def _():
        for c in _copies(i + 1, nxt):
            c.start()

    for c in _copies(i, buf):
        c.wait()

    res = jnp.dot(abuf[buf], vc_ref[...],
                  preferred_element_type=jnp.float32,
                  precision=jax.lax.Precision.DEFAULT)
    num = res[:, :out_f] + vself_ref[:, :out_f]
    den = res[:, out_f:out_f + 1] + vself_ref[:, out_f:out_f + 1]
    out_ref[...] = jax.nn.sigmoid(num / den)


def kernel(X, adj, W, b, S):
    n, in_f = X.shape
    out_f = W.shape[0]

    tp = 1000
    ti = 400
    splits = 10
    nchunks = n // ti

    s2w = S[out_f:].reshape(1, out_f)
    b2 = b.reshape(1, out_f)

    xp, s2, cmax = pl.pallas_call(
        _prologue_body,
        grid=(n // tp,),
        in_specs=[
            pl.BlockSpec((tp, in_f), lambda t: (t, 0)),
            pl.BlockSpec((out_f, in_f), lambda t: (0, 0)),
            pl.BlockSpec((1, out_f), lambda t: (0, 0)),
            pl.BlockSpec((1, out_f), lambda t: (0, 0)),
        ],
        out_specs=[
            pl.BlockSpec((tp, out_f), lambda t: (t, 0)),
            pl.BlockSpec((tp, 1), lambda t: (t, 0)),
            pl.BlockSpec((1, 1), lambda t: (0, 0)),
        ],
        out_shape=[
            jax.ShapeDtypeStruct((n, out_f), jnp.float32),
            jax.ShapeDtypeStruct((n, 1), jnp.float32),
            jax.ShapeDtypeStruct((1, 1), jnp.float32),
        ],
    )(X, W, b2, s2w)

    vc = pl.pallas_call(
        _vbuild_body,
        grid=(n // tp,),
        in_specs=[
            pl.BlockSpec((tp, out_f), lambda t: (t, 0)),
            pl.BlockSpec((tp, 1), lambda t: (t, 0)),
            pl.BlockSpec((1, 1), lambda t: (0, 0)),
        ],
        out_specs=pl.BlockSpec((tp, 2 * out_f), lambda t: (t, 0)),
        out_shape=jax.ShapeDtypeStruct((n, 2 * out_f), jnp.float32),
    )(xp, s2, cmax)

    out = pl.pallas_call(
        functools.partial(_main_body, out_f=out_f, ti=ti, splits=splits,
                          nchunks=nchunks),
        grid=(nchunks,),
        in_specs=[
            pl.BlockSpec(memory_space=pl.ANY),
            pl.BlockSpec((n, 2 * out_f), lambda i: (0, 0)),
            pl.BlockSpec((ti, 2 * out_f), lambda i: (i, 0)),
        ],
        out_specs=pl.BlockSpec((ti, out_f), lambda i: (i, 0)),
        out_shape=jax.ShapeDtypeStruct((n, out_f), jnp.float32),
        scratch_shapes=[
            pltpu.VMEM((2, ti, n), jnp.float32),
            pltpu.SemaphoreType.DMA((2, splits)),
        ],
    )(adj, vc, vc)

    return out
